# baseline (speedup 1.0000x reference)
import functools

import jax
import jax.numpy as jnp
from jax import lax
from jax.experimental import pallas as pl
from jax.experimental.pallas import tpu as pltpu
from jax.experimental.pallas import tpu_sc as plsc

INPUT_SIZE = 200
EMBED = 64
PADDED = 128
LANES = 16
NUM_WORKERS = 32
NBUF = 2
CHUNK_A = 128
CHUNK_B = INPUT_SIZE - CHUNK_A


def _pos_encoding(n=10000):
    pos = jnp.arange(INPUT_SIZE, dtype=jnp.float32)[:, None]
    i = jnp.arange(EMBED // 2, dtype=jnp.float32)
    den = jnp.power(jnp.float32(n), 2.0 * i / EMBED)
    P = jnp.zeros((INPUT_SIZE, EMBED), dtype=jnp.float32)
    P = P.at[:, 0::2].set(jnp.sin(pos / den))
    P = P.at[:, 1::2].set(jnp.cos(pos / den))
    return P


@functools.lru_cache(maxsize=None)
def _build_format(vocab):
    n_cols = (vocab + PADDED - 1) // PADDED
    iters = (n_cols + NUM_WORKERS - 1) // NUM_WORKERS
    last_col = n_cols - 1
    last_vocab = vocab - last_col * PADDED
    mesh = plsc.VectorSubcoreMesh(core_axis_name="c", subcore_axis_name="s")

    @functools.partial(
        pl.kernel,
        mesh=mesh,
        compiler_params=pltpu.CompilerParams(use_tc_tiling_on_sc=True,
                                             needs_layout_passes=False),
        out_type=jax.ShapeDtypeStruct((vocab, PADDED), jnp.float32),
        scratch_types=[
            pltpu.VMEM((NBUF, EMBED, PADDED), jnp.float32),
            pltpu.VMEM((NBUF, PADDED, PADDED), jnp.float32),
        ] + [pltpu.SemaphoreType.DMA] * (2 * NBUF),
    )
    def fmt(tt_hbm, tail_hbm, out_hbm, in_v, out_v, *sems):
        isems, osems = sems[:NBUF], sems[NBUF:]
        wid = lax.axis_index("s") * 2 + lax.axis_index("c")
        iota = lax.iota(jnp.int32, LANES)

        def col_of(t):
            return wid + t * NUM_WORKERS

        def fire_in(c, b):
            pltpu.async_copy(
                tt_hbm.at[pl.ds(0, EMBED), pl.ds(c * PADDED, PADDED)],
                in_v.at[b], isems[b])

        def wait_in(b):
            pltpu.make_async_copy(
                tt_hbm.at[pl.ds(0, EMBED), pl.ds(0, PADDED)],
                in_v.at[b], isems[b]).wait()

        def fire_out(c, b, h):
            pltpu.async_copy(
                out_v.at[b, pl.ds(0, h)],
                out_hbm.at[pl.ds(c * PADDED, h)], osems[b])

        def wait_out(b, h):
            pltpu.make_async_copy(
                out_v.at[b, pl.ds(0, h)], out_hbm.at[pl.ds(0, h)],
                osems[b]).wait()

        def transpose(b, h):
            @plsc.parallel_loop(0, EMBED, unroll=2)
            def _(e):
                evec = jnp.full((LANES,), e, jnp.int32)
                for t in range(h // LANES):
                    rowv = t * LANES + iota
                    v = in_v.at[b][e, pl.ds(t * LANES, LANES)]
                    plsc.store_scatter(out_v.at[b], [rowv, evec], v)
                    plsc.store_scatter(out_v.at[b], [rowv, evec + EMBED], v)

        full_cols = n_cols - 1 if last_vocab < PADDED else n_cols

        for b in range(NBUF):
            @pl.when(col_of(b) < full_cols)
            def _():
                fire_in(col_of(b), b)

        def step(t, carry):
            for b in range(NBUF):
                tt = t * NBUF + b
                c = col_of(tt)

                @pl.when(c < full_cols)
                def _():
                    wait_in(b)

                    @pl.when(tt >= NBUF)
                    def _():
                        wait_out(b, PADDED)

                    transpose(b, PADDED)
                    fire_out(c, b, PADDED)
                    nxt = col_of(tt + NBUF)

                    @pl.when(nxt < full_cols)
                    def _():
                        fire_in(nxt, b)
            return carry

        lax.fori_loop(0, iters // NBUF + 1, step, 0)
        n_valid = (full_cols - wid + NUM_WORKERS - 1) // NUM_WORKERS
        for b in range(NBUF):
            last_t = n_valid - 1 - ((n_valid - 1 - b) % NBUF)

            @pl.when(last_t >= 0)
            def _():
                wait_out(b, PADDED)

        if last_vocab < PADDED:
            @pl.when(wid == last_col % NUM_WORKERS)
            def _():
                pltpu.async_copy(tail_hbm, in_v.at[0], isems[0])
                wait_in(0)
                transpose(0, last_vocab)
                fire_out(last_col, 0, last_vocab)
                wait_out(0, last_vocab)

    return fmt


@functools.lru_cache(maxsize=None)
def _build_gather(n_rows, vocab):
    rows_w = n_rows // NUM_WORKERS
    seqs_w = rows_w // INPUT_SIZE
    n_groups = seqs_w // NBUF
    batch = n_rows // INPUT_SIZE
    mesh = plsc.VectorSubcoreMesh(core_axis_name="c", subcore_axis_name="s")

    @functools.partial(
        pl.kernel,
        mesh=mesh,
        compiler_params=pltpu.CompilerParams(use_tc_tiling_on_sc=True,
                                             needs_layout_passes=False),
        out_type=jax.ShapeDtypeStruct((batch, INPUT_SIZE, PADDED),
                                      jnp.float32),
        scratch_types=[
            pltpu.VMEM((rows_w,), jnp.int32),
            pltpu.VMEM((INPUT_SIZE, EMBED), jnp.float32),
            pltpu.VMEM((NBUF, INPUT_SIZE, PADDED), jnp.float32),
        ] + [pltpu.SemaphoreType.DMA] * (2 * NBUF),
    )
    def gather_add(pairs_hbm, idx_hbm, p_hbm, out_hbm, idx_v, p_v, rows_v,
                   *sems):
        gsems, osems = sems[:NBUF], sems[NBUF:]
        wid = lax.axis_index("s") * 2 + lax.axis_index("c")
        base = pl.multiple_of(wid * rows_w, 8)
        seq0 = wid * seqs_w
        pltpu.sync_copy(idx_hbm.at[pl.ds(base, rows_w)], idx_v)
        pltpu.sync_copy(p_hbm, p_v)

        def fire_gather(s, b):
            row0 = pl.multiple_of(s * INPUT_SIZE, 8)
            pltpu.async_copy(
                pairs_hbm.at[idx_v.at[pl.ds(row0, CHUNK_A)]],
                rows_v.at[b, pl.ds(0, CHUNK_A)], gsems[b])
            pltpu.async_copy(
                pairs_hbm.at[idx_v.at[pl.ds(row0 + CHUNK_A, CHUNK_B)]],
                rows_v.at[b, pl.ds(CHUNK_A, CHUNK_B)], gsems[b])

        def wait_gather(b):
            pltpu.make_async_copy(
                pairs_hbm.at[pl.ds(0, INPUT_SIZE)], rows_v.at[b],
                gsems[b]).wait()

        def fire_out(s, b):
            pltpu.async_copy(rows_v.at[b], out_hbm.at[seq0 + s], osems[b])

        def wait_out(b):
            pltpu.make_async_copy(
                rows_v.at[b], out_hbm.at[0], osems[b]).wait()

        for b in range(NBUF):
            fire_gather(b, b)

        def group(g, carry):
            for b in range(NBUF):
                s = g * NBUF + b
                wait_gather(b)

                @plsc.parallel_loop(0, INPUT_SIZE, unroll=4)
                def _(r):
                    for j in range(EMBED // LANES):
                        sl = pl.ds(j * LANES, LANES)
                        plsc.addupdate(rows_v.at[b, r, sl], p_v[r, sl])

                fire_out(s, b)

            @pl.when(g + 1 < n_groups)
            def _():
                for b in range(NBUF):
                    wait_out(b)
                    fire_gather((g + 1) * NBUF + b, b)

            return carry

        lax.fori_loop(0, n_groups, group, 0)
        for b in range(NBUF):
            wait_out(b)

    return gather_add


def kernel(x, table):
    b, l = x.shape
    idx = x.reshape(-1)
    if idx.dtype != jnp.int32:
        idx = idx.astype(jnp.int32)
    vocab = table.shape[0]
    n_full = (vocab // PADDED) * PADDED
    tail = jnp.zeros((EMBED, PADDED), jnp.float32)
    if n_full < vocab:
        tail = tail.at[:, : vocab - n_full].set(table[n_full:].T)
    pairs = _build_format(vocab)(table.T, tail)
    p = _pos_encoding()
    out = _build_gather(b * l, vocab)(pairs, idx, p)
    return out[:, :, :EMBED]


# speedup vs baseline: 3.0684x; 3.0684x over previous
import functools

import jax
import jax.numpy as jnp
from jax import lax
from jax.experimental import pallas as pl
from jax.experimental.pallas import tpu as pltpu
from jax.experimental.pallas import tpu_sc as plsc

INPUT_SIZE = 200
EMBED = 64
PADDED = 128
LANES = 16
NUM_WORKERS = 32
NBUF = 2
CHUNK_A = 128
CHUNK_B = INPUT_SIZE - CHUNK_A


def _pos_encoding(n=10000):
    pos = jnp.arange(INPUT_SIZE, dtype=jnp.float32)[:, None]
    i = jnp.arange(EMBED // 2, dtype=jnp.float32)
    den = jnp.power(jnp.float32(n), 2.0 * i / EMBED)
    P = jnp.zeros((INPUT_SIZE, EMBED), dtype=jnp.float32)
    P = P.at[:, 0::2].set(jnp.sin(pos / den))
    P = P.at[:, 1::2].set(jnp.cos(pos / den))
    return P


@functools.lru_cache(maxsize=None)
def _build_gather(n_rows, vocab):
    rows_w = n_rows // NUM_WORKERS
    seqs_w = rows_w // INPUT_SIZE
    n_groups = seqs_w // NBUF
    batch = n_rows // INPUT_SIZE
    mesh = plsc.VectorSubcoreMesh(core_axis_name="c", subcore_axis_name="s")

    @functools.partial(
        pl.kernel,
        mesh=mesh,
        compiler_params=pltpu.CompilerParams(use_tc_tiling_on_sc=True),
        out_type=jax.ShapeDtypeStruct((batch, INPUT_SIZE, PADDED),
                                      jnp.float32),
        scratch_types=[
            pltpu.VMEM((rows_w,), jnp.int32),
            pltpu.VMEM((INPUT_SIZE, PADDED), jnp.float32),
            pltpu.VMEM((NBUF, INPUT_SIZE, PADDED), jnp.float32),
        ] + [pltpu.SemaphoreType.DMA] * (2 * NBUF),
    )
    def gather_add(table_hbm, idx_hbm, p_hbm, out_hbm, idx_v, p_v, rows_v,
                   *sems):
        gsems, osems = sems[:NBUF], sems[NBUF:]
        wid = lax.axis_index("s") * 2 + lax.axis_index("c")
        base = pl.multiple_of(wid * rows_w, 8)
        seq0 = wid * seqs_w
        pltpu.sync_copy(idx_hbm.at[pl.ds(base, rows_w)], idx_v)
        pltpu.sync_copy(p_hbm, p_v)

        def fire_gather(s, b):
            row0 = pl.multiple_of(s * INPUT_SIZE, 8)
            pltpu.async_copy(
                table_hbm.at[idx_v.at[pl.ds(row0, CHUNK_A)]],
                rows_v.at[b, pl.ds(0, CHUNK_A)], gsems[b])
            pltpu.async_copy(
                table_hbm.at[idx_v.at[pl.ds(row0 + CHUNK_A, CHUNK_B)]],
                rows_v.at[b, pl.ds(CHUNK_A, CHUNK_B)], gsems[b])

        def wait_gather(b):
            pltpu.make_async_copy(
                table_hbm.at[pl.ds(0, INPUT_SIZE)], rows_v.at[b],
                gsems[b]).wait()

        def fire_out(s, b):
            pltpu.async_copy(rows_v.at[b], out_hbm.at[seq0 + s], osems[b])

        def wait_out(b):
            pltpu.make_async_copy(
                rows_v.at[b], out_hbm.at[0], osems[b]).wait()

        for b in range(NBUF):
            fire_gather(b, b)

        def group(g, carry):
            for b in range(NBUF):
                s = g * NBUF + b
                wait_gather(b)

                @plsc.parallel_loop(0, INPUT_SIZE, unroll=4)
                def _(r):
                    for j in range(EMBED // LANES):
                        sl = pl.ds(j * LANES, LANES)
                        plsc.addupdate(rows_v.at[b, r, sl], p_v[r, sl])

                fire_out(s, b)

            @pl.when(g + 1 < n_groups)
            def _():
                for b in range(NBUF):
                    wait_out(b)
                    fire_gather((g + 1) * NBUF + b, b)

            return carry

        lax.fori_loop(0, n_groups, group, 0)
        for b in range(NBUF):
            wait_out(b)

    return gather_add


def kernel(x, table):
    b, l = x.shape
    idx = x.reshape(-1)
    if idx.dtype != jnp.int32:
        idx = idx.astype(jnp.int32)
    table128 = jnp.pad(table, ((0, 0), (0, PADDED - EMBED)))
    p = jnp.pad(_pos_encoding(), ((0, 0), (0, PADDED - EMBED)))
    out = _build_gather(b * l, table.shape[0])(table128, idx, p)
    return out[:, :, :EMBED]
